# baseline (speedup 1.0000x reference)
import jax
import jax.numpy as jnp
from jax import lax
from jax.experimental import pallas as pl
from jax.experimental.pallas import tpu as pltpu
from jax.experimental.pallas import tpu_sc as plsc

N = 10000
E = 320000
D = 128
D_EA = 128
NC = 2
NS = 16
CH = 128
NW = NC * NS
CHUNKS = -(-E // (CH * NW))
EP = CHUNKS * CH * NW
CHUNKS_A = 153
CHUNKS_B = 2 * CHUNKS - CHUNKS_A
ACC_ROWS = 10112
ZROWS = ACC_ROWS // NS
WROWS = 624
WTAIL = N - NS * WROWS

import functools


@functools.cache
def _mesh():
    return plsc.VectorSubcoreMesh(core_axis_name="c", subcore_axis_name="s")


def _writeback(acc, out_hbm, c, s):
    r0 = s * WROWS
    pltpu.sync_copy(acc.at[pl.ds(r0, WROWS)],
                    out_hbm.at[pl.ds(c * N + r0, WROWS)])

    @pl.when(s == 0)
    def _():
        t0 = NS * WROWS
        pltpu.sync_copy(acc.at[pl.ds(t0, WTAIL)],
                        out_hbm.at[pl.ds(c * N + t0, WTAIL)])


def _gather_seg_sum_body(x_hbm, sd_hbm, z_hbm, px_hbm,
                         acc, sda, sdb, rows0, rows1, sem0, sem1, isa, isb):
    c = lax.axis_index("c")
    s = lax.axis_index("s")

    pltpu.sync_copy(z_hbm, acc.at[pl.ds(s * ZROWS, ZROWS)])
    plsc.subcore_barrier()

    cnt = jnp.where(c == 0, CHUNKS_A, CHUNKS_B)
    base = jnp.where(c == 0, s * CHUNKS_A,
                     NS * CHUNKS_A + s * CHUNKS_B)

    pltpu.sync_copy(sd_hbm.at[pl.ds(base, 1)], sdb.at[pl.ds(0, 1)])
    pltpu.async_copy(x_hbm.at[sdb.at[0, 0]], rows0, sem0).wait()
    pltpu.sync_copy(rows0, acc.at[sdb.at[0, 1]], add=True)
    pltpu.async_copy(sd_hbm.at[pl.ds(base + 1, 2)], sda, isa)

    @pl.loop(1, cnt, step=4)
    def _(i):
        pltpu.make_async_copy(sd_hbm.at[pl.ds(0, 2)], sda, isa).wait()
        d0 = pltpu.async_copy(x_hbm.at[sda.at[0, 0]], rows0, sem0)
        d1 = pltpu.async_copy(x_hbm.at[sda.at[1, 0]], rows1, sem1)
        pltpu.async_copy(sd_hbm.at[pl.ds(i + base + 2, 2)], sdb, isb)
        d0.wait()
        pltpu.sync_copy(rows0, acc.at[sda.at[0, 1]], add=True)
        d1.wait()
        pltpu.sync_copy(rows1, acc.at[sda.at[1, 1]], add=True)
        pltpu.make_async_copy(sd_hbm.at[pl.ds(0, 2)], sdb, isb).wait()
        e0 = pltpu.async_copy(x_hbm.at[sdb.at[0, 0]], rows0, sem0)
        e1 = pltpu.async_copy(x_hbm.at[sdb.at[1, 0]], rows1, sem1)
        pltpu.async_copy(sd_hbm.at[pl.ds(i + base + 4, 2)], sda, isa)
        e0.wait()
        pltpu.sync_copy(rows0, acc.at[sdb.at[0, 1]], add=True)
        e1.wait()
        pltpu.sync_copy(rows1, acc.at[sdb.at[1, 1]], add=True)

    pltpu.make_async_copy(sd_hbm.at[pl.ds(0, 2)], sda, isa).wait()

    plsc.subcore_barrier()
    _writeback(acc, px_hbm, c, s)


@functools.cache
def _gather_seg_sum():
    return pl.kernel(
        _gather_seg_sum_body,
        out_type=jax.ShapeDtypeStruct((NC * N, D), jnp.float32),
        mesh=_mesh(),
        scratch_types=[
            pltpu.VMEM_SHARED((ACC_ROWS, D), jnp.float32),
            pltpu.VMEM((2, 2, CH), jnp.int32),
            pltpu.VMEM((2, 2, CH), jnp.int32),
            pltpu.VMEM((CH, D), jnp.float32),
            pltpu.VMEM((CH, D), jnp.float32),
            pltpu.SemaphoreType.DMA,
            pltpu.SemaphoreType.DMA,
            pltpu.SemaphoreType.DMA,
            pltpu.SemaphoreType.DMA,
        ],
    )


def _ea_seg_sum_body(ea_hbm, sd_hbm, z_hbm, pe_hbm,
                     acc, sdp, ea0, ea1, sem0, sem1):
    c = lax.axis_index("c")
    s = lax.axis_index("s")
    wid = s * NC + c

    pltpu.sync_copy(z_hbm, acc.at[pl.ds(s * ZROWS, ZROWS)])
    plsc.subcore_barrier()

    base = wid * CHUNKS

    e00 = pl.multiple_of(base * CH, CH)
    pltpu.sync_copy(sd_hbm.at[pl.ds(base, 1)], sdp.at[pl.ds(0, 1)])
    pltpu.async_copy(ea_hbm.at[pl.ds(e00, CH)], ea0, sem0).wait()
    pltpu.sync_copy(ea0, acc.at[sdp.at[0, 1]], add=True)

    @pl.loop(1, CHUNKS, step=2)
    def _(i):
        e0 = pl.multiple_of((base + i) * CH, CH)
        pltpu.sync_copy(sd_hbm.at[pl.ds(i + base, 2)], sdp)
        d0 = pltpu.async_copy(ea_hbm.at[pl.ds(e0, CH)], ea0, sem0)
        d1 = pltpu.async_copy(ea_hbm.at[pl.ds(e0 + CH, CH)], ea1, sem1)
        d0.wait()
        pltpu.sync_copy(ea0, acc.at[sdp.at[0, 1]], add=True)
        d1.wait()
        pltpu.sync_copy(ea1, acc.at[sdp.at[1, 1]], add=True)

    plsc.subcore_barrier()
    _writeback(acc, pe_hbm, c, s)


@functools.cache
def _ea_seg_sum():
    return pl.kernel(
        _ea_seg_sum_body,
        out_type=jax.ShapeDtypeStruct((NC * N, D_EA), jnp.float32),
        mesh=_mesh(),
        scratch_types=[
            pltpu.VMEM_SHARED((ACC_ROWS, D_EA), jnp.float32),
            pltpu.VMEM((2, 2, CH), jnp.int32),
            pltpu.VMEM((CH, D_EA), jnp.float32),
            pltpu.VMEM((CH, D_EA), jnp.float32),
            pltpu.SemaphoreType.DMA,
            pltpu.SemaphoreType.DMA,
        ],
    )


BLK = 1000


def _dense_body(xin, p0, p1, q0, q1, wn, we, bnbe, wua, wub, bu, g, beta,
                out):
    sx = p0[...] + p1[...]
    se = q0[...] + q1[...]
    cnt = se[:, 16:17]
    rinv = 1.0 / jnp.maximum(cnt, 1.0)
    aggr = (jnp.dot(sx, wn[...], preferred_element_type=jnp.float32)
            + jnp.dot(se[:, :16], we[...], preferred_element_type=jnp.float32)
            + cnt * bnbe[...]) * rinv
    h = (jnp.dot(xin[...], wua[...], preferred_element_type=jnp.float32)
         + jnp.dot(aggr, wub[...], preferred_element_type=jnp.float32)
         + bu[...])
    h = jnp.maximum(h, 0.0)
    mu = jnp.mean(h, axis=-1, keepdims=True)
    var = jnp.mean((h - mu) ** 2, axis=-1, keepdims=True)
    out[...] = (h - mu) * jax.lax.rsqrt(var + 1e-5) * g[...] + beta[...]


def _dense_layer(xin, px, pe, Wn, We, bnbe, Wua, Wub, bu, g, beta):
    nb = N // BLK
    full = lambda i: (0, 0)
    return pl.pallas_call(
        _dense_body,
        grid=(nb,),
        in_specs=[
            pl.BlockSpec((BLK, D), lambda i: (i, 0)),
            pl.BlockSpec((BLK, D), lambda i: (i, 0)),
            pl.BlockSpec((BLK, D), lambda i, _n=nb: (i + _n, 0)),
            pl.BlockSpec((BLK, D_EA), lambda i: (i, 0)),
            pl.BlockSpec((BLK, D_EA), lambda i, _n=nb: (i + _n, 0)),
            pl.BlockSpec((D, D), full),
            pl.BlockSpec((16, D), full),
            pl.BlockSpec((1, D), full),
            pl.BlockSpec((D, D), full),
            pl.BlockSpec((D, D), full),
            pl.BlockSpec((1, D), full),
            pl.BlockSpec((1, D), full),
            pl.BlockSpec((1, D), full),
        ],
        out_specs=pl.BlockSpec((BLK, D), lambda i: (i, 0)),
        out_shape=jax.ShapeDtypeStruct((N, D), jnp.float32),
    )(xin, px, px, pe, pe, Wn, We, bnbe, Wua, Wub, bu, g, beta)


def kernel(x, edge_index, edge_attr, Wn1, bn1, We1, be1, Wu1, bu1, g1, beta1,
           Wn2, bn2, We2, be2, Wu2, bu2, g2, beta2):
    src = edge_index[0]
    dst = edge_index[1]
    pad = EP - E
    src_p = jnp.concatenate([src, jnp.zeros((pad,), jnp.int32)])
    dst_p = jnp.concatenate([dst, jnp.full((pad,), N, jnp.int32)])
    sd = jnp.stack([src_p.reshape(-1, CH), dst_p.reshape(-1, CH)], axis=1)
    sd = jnp.concatenate([sd, jnp.zeros((2, 2, CH), jnp.int32)], axis=0)
    ea = jnp.concatenate(
        [edge_attr,
         jnp.ones((E, 1), jnp.float32),
         jnp.zeros((E, D_EA - 17), jnp.float32)], axis=1)
    ea = jnp.concatenate([ea, jnp.zeros((pad, D_EA), jnp.float32)], axis=0)
    zx = jnp.zeros((ZROWS, D), jnp.float32)

    pe = _ea_seg_sum()(ea, sd, zx)
    px = _gather_seg_sum()(x, sd, zx)
    h1 = _dense_layer(x, px, pe, Wn1, We1, (bn1 + be1).reshape(1, D),
                      Wu1[:D], Wu1[D:], bu1.reshape(1, D),
                      g1.reshape(1, D), beta1.reshape(1, D))
    ph = _gather_seg_sum()(h1, sd, zx)
    out = _dense_layer(h1, ph, pe, Wn2, We2, (bn2 + be2).reshape(1, D),
                       Wu2[:D], Wu2[D:], bu2.reshape(1, D),
                       g2.reshape(1, D), beta2.reshape(1, D))
    return out


# speedup vs baseline: 1.0657x; 1.0657x over previous
import jax
import jax.numpy as jnp
from jax import lax
from jax.experimental import pallas as pl
from jax.experimental.pallas import tpu as pltpu
from jax.experimental.pallas import tpu_sc as plsc

N = 10000
E = 320000
D = 128
D_EA = 128
NC = 2
NS = 16
CH = 128
NW = NC * NS
CHUNKS = -(-E // (CH * NW))
EP = CHUNKS * CH * NW
CHUNKS_A = 129
CHUNKS_B = 2 * CHUNKS - CHUNKS_A
ACC_ROWS = 10112
ZROWS = ACC_ROWS // NS
WROWS = 624
WTAIL = N - NS * WROWS

import functools


@functools.cache
def _mesh():
    return plsc.VectorSubcoreMesh(core_axis_name="c", subcore_axis_name="s")


def _writeback(acc, out_hbm, c, s):
    r0 = s * WROWS
    pltpu.sync_copy(acc.at[pl.ds(r0, WROWS)],
                    out_hbm.at[pl.ds(c * N + r0, WROWS)])

    @pl.when(s == 0)
    def _():
        t0 = NS * WROWS
        pltpu.sync_copy(acc.at[pl.ds(t0, WTAIL)],
                        out_hbm.at[pl.ds(c * N + t0, WTAIL)])


def _gather_seg_sum_body(x_hbm, sd_hbm, z_hbm, px_hbm,
                         acc, sda, sdb, rows0, rows1, sem0, sem1, isa, isb):
    c = lax.axis_index("c")
    s = lax.axis_index("s")

    pltpu.sync_copy(z_hbm, acc.at[pl.ds(s * ZROWS, ZROWS)])
    plsc.subcore_barrier()

    cnt = jnp.where(c == 0, CHUNKS_A, CHUNKS_B)
    base = jnp.where(c == 0, s * CHUNKS_A,
                     NS * CHUNKS_A + s * CHUNKS_B)

    pltpu.sync_copy(sd_hbm.at[pl.ds(base, 1)], sdb.at[pl.ds(0, 1)])
    pltpu.async_copy(x_hbm.at[sdb.at[0, 0]], rows0, sem0).wait()
    pltpu.sync_copy(rows0, acc.at[sdb.at[0, 1]], add=True)
    pltpu.async_copy(sd_hbm.at[pl.ds(base + 1, 2)], sda, isa)

    @pl.loop(1, cnt, step=4)
    def _(i):
        pltpu.make_async_copy(sd_hbm.at[pl.ds(0, 2)], sda, isa).wait()
        d0 = pltpu.async_copy(x_hbm.at[sda.at[0, 0]], rows0, sem0)
        d1 = pltpu.async_copy(x_hbm.at[sda.at[1, 0]], rows1, sem1)
        pltpu.async_copy(sd_hbm.at[pl.ds(i + base + 2, 2)], sdb, isb)
        d0.wait()
        pltpu.sync_copy(rows0, acc.at[sda.at[0, 1]], add=True)
        d1.wait()
        pltpu.sync_copy(rows1, acc.at[sda.at[1, 1]], add=True)
        pltpu.make_async_copy(sd_hbm.at[pl.ds(0, 2)], sdb, isb).wait()
        e0 = pltpu.async_copy(x_hbm.at[sdb.at[0, 0]], rows0, sem0)
        e1 = pltpu.async_copy(x_hbm.at[sdb.at[1, 0]], rows1, sem1)
        pltpu.async_copy(sd_hbm.at[pl.ds(i + base + 4, 2)], sda, isa)
        e0.wait()
        pltpu.sync_copy(rows0, acc.at[sdb.at[0, 1]], add=True)
        e1.wait()
        pltpu.sync_copy(rows1, acc.at[sdb.at[1, 1]], add=True)

    pltpu.make_async_copy(sd_hbm.at[pl.ds(0, 2)], sda, isa).wait()

    plsc.subcore_barrier()
    _writeback(acc, px_hbm, c, s)


@functools.cache
def _gather_seg_sum():
    return pl.kernel(
        _gather_seg_sum_body,
        out_type=jax.ShapeDtypeStruct((NC * N, D), jnp.float32),
        mesh=_mesh(),
        scratch_types=[
            pltpu.VMEM_SHARED((ACC_ROWS, D), jnp.float32),
            pltpu.VMEM((2, 2, CH), jnp.int32),
            pltpu.VMEM((2, 2, CH), jnp.int32),
            pltpu.VMEM((CH, D), jnp.float32),
            pltpu.VMEM((CH, D), jnp.float32),
            pltpu.SemaphoreType.DMA,
            pltpu.SemaphoreType.DMA,
            pltpu.SemaphoreType.DMA,
            pltpu.SemaphoreType.DMA,
        ],
    )


def _ea_seg_sum_body(ea_hbm, sd_hbm, z_hbm, pe_hbm,
                     acc, sdp, ea0, ea1, sem0, sem1):
    c = lax.axis_index("c")
    s = lax.axis_index("s")
    wid = s * NC + c

    pltpu.sync_copy(z_hbm, acc.at[pl.ds(s * ZROWS, ZROWS)])
    plsc.subcore_barrier()

    base = wid * CHUNKS

    e00 = pl.multiple_of(base * CH, CH)
    pltpu.sync_copy(sd_hbm.at[pl.ds(base, 1)], sdp.at[pl.ds(0, 1)])
    pltpu.async_copy(ea_hbm.at[pl.ds(e00, CH)], ea0, sem0).wait()
    pltpu.sync_copy(ea0, acc.at[sdp.at[0, 1]], add=True)

    @pl.loop(1, CHUNKS, step=2)
    def _(i):
        e0 = pl.multiple_of((base + i) * CH, CH)
        pltpu.sync_copy(sd_hbm.at[pl.ds(i + base, 2)], sdp)
        d0 = pltpu.async_copy(ea_hbm.at[pl.ds(e0, CH)], ea0, sem0)
        d1 = pltpu.async_copy(ea_hbm.at[pl.ds(e0 + CH, CH)], ea1, sem1)
        d0.wait()
        pltpu.sync_copy(ea0, acc.at[sdp.at[0, 1]], add=True)
        d1.wait()
        pltpu.sync_copy(ea1, acc.at[sdp.at[1, 1]], add=True)

    plsc.subcore_barrier()
    _writeback(acc, pe_hbm, c, s)


@functools.cache
def _ea_seg_sum():
    return pl.kernel(
        _ea_seg_sum_body,
        out_type=jax.ShapeDtypeStruct((NC * N, D_EA), jnp.float32),
        mesh=_mesh(),
        scratch_types=[
            pltpu.VMEM_SHARED((ACC_ROWS, D_EA), jnp.float32),
            pltpu.VMEM((2, 2, CH), jnp.int32),
            pltpu.VMEM((CH, D_EA), jnp.float32),
            pltpu.VMEM((CH, D_EA), jnp.float32),
            pltpu.SemaphoreType.DMA,
            pltpu.SemaphoreType.DMA,
        ],
    )


BLK = 1000


def _dense_body(xin, p0, p1, q0, q1, wn, we, bnbe, wua, wub, bu, g, beta,
                out):
    sx = p0[...] + p1[...]
    se = q0[...] + q1[...]
    cnt = se[:, 16:17]
    rinv = 1.0 / jnp.maximum(cnt, 1.0)
    aggr = (jnp.dot(sx, wn[...], preferred_element_type=jnp.float32)
            + jnp.dot(se[:, :16], we[...], preferred_element_type=jnp.float32)
            + cnt * bnbe[...]) * rinv
    h = (jnp.dot(xin[...], wua[...], preferred_element_type=jnp.float32)
         + jnp.dot(aggr, wub[...], preferred_element_type=jnp.float32)
         + bu[...])
    h = jnp.maximum(h, 0.0)
    mu = jnp.mean(h, axis=-1, keepdims=True)
    var = jnp.mean((h - mu) ** 2, axis=-1, keepdims=True)
    out[...] = (h - mu) * jax.lax.rsqrt(var + 1e-5) * g[...] + beta[...]


def _dense_layer(xin, px, pe, Wn, We, bnbe, Wua, Wub, bu, g, beta):
    nb = N // BLK
    full = lambda i: (0, 0)
    return pl.pallas_call(
        _dense_body,
        grid=(nb,),
        in_specs=[
            pl.BlockSpec((BLK, D), lambda i: (i, 0)),
            pl.BlockSpec((BLK, D), lambda i: (i, 0)),
            pl.BlockSpec((BLK, D), lambda i, _n=nb: (i + _n, 0)),
            pl.BlockSpec((BLK, D_EA), lambda i: (i, 0)),
            pl.BlockSpec((BLK, D_EA), lambda i, _n=nb: (i + _n, 0)),
            pl.BlockSpec((D, D), full),
            pl.BlockSpec((16, D), full),
            pl.BlockSpec((1, D), full),
            pl.BlockSpec((D, D), full),
            pl.BlockSpec((D, D), full),
            pl.BlockSpec((1, D), full),
            pl.BlockSpec((1, D), full),
            pl.BlockSpec((1, D), full),
        ],
        out_specs=pl.BlockSpec((BLK, D), lambda i: (i, 0)),
        out_shape=jax.ShapeDtypeStruct((N, D), jnp.float32),
    )(xin, px, px, pe, pe, Wn, We, bnbe, Wua, Wub, bu, g, beta)


def kernel(x, edge_index, edge_attr, Wn1, bn1, We1, be1, Wu1, bu1, g1, beta1,
           Wn2, bn2, We2, be2, Wu2, bu2, g2, beta2):
    src = edge_index[0]
    dst = edge_index[1]
    pad = EP - E
    src_p = jnp.concatenate([src, jnp.zeros((pad,), jnp.int32)])
    dst_p = jnp.concatenate([dst, jnp.full((pad,), N, jnp.int32)])
    sd = jnp.stack([src_p.reshape(-1, CH), dst_p.reshape(-1, CH)], axis=1)
    sd = jnp.concatenate([sd, jnp.zeros((2, 2, CH), jnp.int32)], axis=0)
    ea = jnp.concatenate(
        [edge_attr,
         jnp.ones((E, 1), jnp.float32),
         jnp.zeros((E, D_EA - 17), jnp.float32)], axis=1)
    ea = jnp.concatenate([ea, jnp.zeros((pad, D_EA), jnp.float32)], axis=0)
    zx = jnp.zeros((ZROWS, D), jnp.float32)

    pe = _ea_seg_sum()(ea, sd, zx)
    px = _gather_seg_sum()(x, sd, zx)
    h1 = _dense_layer(x, px, pe, Wn1, We1, (bn1 + be1).reshape(1, D),
                      Wu1[:D], Wu1[D:], bu1.reshape(1, D),
                      g1.reshape(1, D), beta1.reshape(1, D))
    ph = _gather_seg_sum()(h1, sd, zx)
    out = _dense_layer(h1, ph, pe, Wn2, We2, (bn2 + be2).reshape(1, D),
                       Wu2[:D], Wu2[D:], bu2.reshape(1, D),
                       g2.reshape(1, D), beta2.reshape(1, D))
    return out
